# baseline (speedup 1.0000x reference)
import jax
import jax.numpy as jnp
from jax.experimental import pallas as pl
from jax.experimental.pallas import tpu as pltpu

G = 4


def _src_tap(s, i, j):
    if s == 0:
        return (i, j)
    if s == 1:
        return (j, 2 - i)
    if s == 2:
        return (2 - i, 2 - j)
    return (2 - j, i)


def _transform_body(k_ref, b_ref, wt_ref, rb_ref):
    gcin = k_ref.shape[2]
    cin = gcin // G
    for i in range(3):
        for j in range(3):
            cols = []
            for s in range(G):
                si, sj = _src_tap(s, i, j)
                blk = k_ref[si, sj]
                sh = s * cin
                if sh:
                    blk = jnp.concatenate([blk[gcin - sh:], blk[:gcin - sh]],
                                          axis=0)
                cols.append(blk)
            wt_ref[i, j] = jnp.concatenate(cols, axis=1).astype(jnp.bfloat16)
    rb_ref[...] = jnp.concatenate([b_ref[...]] * G, axis=-1)


def _conv_body(x_hbm, wt_ref, rb_ref, out_ref, buf, sems, *, th, w, n_tiles):
    r = pl.program_id(0)
    kdim = x_hbm.shape[2]
    n = wt_ref.shape[3]
    hw = th + 2

    def issue(t):
        slot = jax.lax.rem(t, 2)

        @pl.when(t == 0)
        def _():
            buf[slot, 0] = jnp.zeros((w, kdim), buf.dtype)
            pltpu.make_async_copy(
                x_hbm.at[pl.ds(0, th + 1)],
                buf.at[slot, pl.ds(1, th + 1)],
                sems.at[slot]).start()

        @pl.when(jnp.logical_and(t > 0, t < n_tiles - 1))
        def _():
            pltpu.make_async_copy(
                x_hbm.at[pl.ds(jnp.clip(t * th - 1, 0, x_hbm.shape[0] - hw),
                               hw)],
                buf.at[slot, pl.ds(0, hw)],
                sems.at[slot]).start()

        @pl.when(t == n_tiles - 1)
        def _():
            buf[slot, hw - 1] = jnp.zeros((w, kdim), buf.dtype)
            pltpu.make_async_copy(
                x_hbm.at[pl.ds((n_tiles - 1) * th - 1, th + 1)],
                buf.at[slot, pl.ds(0, th + 1)],
                sems.at[slot]).start()

    def wait(t):
        slot = jax.lax.rem(t, 2)
        edge = jnp.logical_or(t == 0, t == n_tiles - 1)

        @pl.when(edge)
        def _():
            pltpu.make_async_copy(
                x_hbm.at[pl.ds(0, th + 1)],
                buf.at[slot, pl.ds(1, th + 1)],
                sems.at[slot]).wait()

        @pl.when(jnp.logical_not(edge))
        def _():
            pltpu.make_async_copy(
                x_hbm.at[pl.ds(0, hw)],
                buf.at[slot, pl.ds(0, hw)],
                sems.at[slot]).wait()

    @pl.when(r == 0)
    def _():
        issue(0)

    @pl.when(r + 1 < n_tiles)
    def _():
        issue(r + 1)

    wait(r)

    slot = jax.lax.rem(r, 2)
    xc = buf[slot].astype(jnp.bfloat16)
    zcol = jnp.zeros((hw, 1, kdim), jnp.bfloat16)
    srcs = (
        jnp.concatenate([zcol, xc[:, :w - 1]], axis=1),
        xc,
        jnp.concatenate([xc[:, 1:], zcol], axis=1),
    )
    acc = jnp.zeros((th * w, n), jnp.float32)
    for j in range(3):
        for i in range(3):
            a = srcs[j][i:i + th].reshape(th * w, kdim)
            acc += jnp.dot(a, wt_ref[i, j], preferred_element_type=jnp.float32)
    out_ref[...] = (acc + rb_ref[0, :][None, :]).reshape(th, w, n)


def kernel(inputs, kernel, bias):
    B, H, W, G_, C = inputs.shape
    kh, kw, gcin, cout = kernel.shape
    n_out = G * cout

    x = inputs.reshape(H, W, G_ * C)

    wt, rb = pl.pallas_call(
        _transform_body,
        out_shape=(
            jax.ShapeDtypeStruct((kh, kw, gcin, n_out), jnp.bfloat16),
            jax.ShapeDtypeStruct((1, n_out), jnp.float32),
        ),
    )(kernel, bias.reshape(1, cout))

    TH = 16
    n_tiles = H // TH

    conv = pl.pallas_call(
        lambda x_hbm, wt_ref, rb_ref, out_ref, buf, sems: _conv_body(
            x_hbm, wt_ref, rb_ref, out_ref, buf, sems,
            th=TH, w=W, n_tiles=n_tiles),
        grid=(n_tiles,),
        in_specs=[
            pl.BlockSpec(memory_space=pltpu.MemorySpace.HBM),
            pl.BlockSpec((kh, kw, gcin, n_out), lambda r: (0, 0, 0, 0)),
            pl.BlockSpec((1, n_out), lambda r: (0, 0)),
        ],
        out_specs=pl.BlockSpec((TH, W, n_out), lambda r: (r, 0, 0)),
        out_shape=jax.ShapeDtypeStruct((H, W, n_out), jnp.float32),
        scratch_shapes=[
            pltpu.VMEM((2, TH + 2, W, gcin), jnp.float32),
            pltpu.SemaphoreType.DMA((2,)),
        ],
        compiler_params=pltpu.CompilerParams(
            dimension_semantics=("arbitrary",),
        ),
    )(x, wt, rb)

    return conv.reshape(B, H, W, G, cout)


# speedup vs baseline: 1.0002x; 1.0002x over previous
import jax
import jax.numpy as jnp
from jax.experimental import pallas as pl
from jax.experimental.pallas import tpu as pltpu

G = 4


def _src_tap(s, i, j):
    if s == 0:
        return (i, j)
    if s == 1:
        return (j, 2 - i)
    if s == 2:
        return (2 - i, 2 - j)
    return (2 - j, i)


def _transform_body(k_ref, b_ref, wt_ref, rb_ref):
    gcin = k_ref.shape[2]
    cin = gcin // G
    for i in range(3):
        for j in range(3):
            cols = []
            for s in range(G):
                si, sj = _src_tap(s, i, j)
                blk = k_ref[si, sj]
                sh = s * cin
                if sh:
                    blk = jnp.concatenate([blk[gcin - sh:], blk[:gcin - sh]],
                                          axis=0)
                cols.append(blk)
            wt_ref[i, j] = jnp.concatenate(cols, axis=1).astype(jnp.bfloat16)
    rb_ref[...] = jnp.concatenate([b_ref[...]] * G, axis=-1)


def _conv_body(x_hbm, wt_ref, rb_ref, out_ref, buf, sems, *, th, w, n_tiles):
    r = pl.program_id(0)
    kdim = x_hbm.shape[2]
    n = wt_ref.shape[3]
    hw = th + 2

    def issue(t):
        slot = jax.lax.rem(t, 2)

        @pl.when(t == 0)
        def _():
            buf[slot, 0] = jnp.zeros((w, kdim), buf.dtype)
            pltpu.make_async_copy(
                x_hbm.at[pl.ds(0, th + 1)],
                buf.at[slot, pl.ds(1, th + 1)],
                sems.at[slot]).start()

        @pl.when(jnp.logical_and(t > 0, t < n_tiles - 1))
        def _():
            pltpu.make_async_copy(
                x_hbm.at[pl.ds(jnp.clip(t * th - 1, 0, x_hbm.shape[0] - hw),
                               hw)],
                buf.at[slot, pl.ds(0, hw)],
                sems.at[slot]).start()

        @pl.when(t == n_tiles - 1)
        def _():
            buf[slot, hw - 1] = jnp.zeros((w, kdim), buf.dtype)
            pltpu.make_async_copy(
                x_hbm.at[pl.ds((n_tiles - 1) * th - 1, th + 1)],
                buf.at[slot, pl.ds(0, th + 1)],
                sems.at[slot]).start()

    def wait(t):
        slot = jax.lax.rem(t, 2)
        edge = jnp.logical_or(t == 0, t == n_tiles - 1)

        @pl.when(edge)
        def _():
            pltpu.make_async_copy(
                x_hbm.at[pl.ds(0, th + 1)],
                buf.at[slot, pl.ds(1, th + 1)],
                sems.at[slot]).wait()

        @pl.when(jnp.logical_not(edge))
        def _():
            pltpu.make_async_copy(
                x_hbm.at[pl.ds(0, hw)],
                buf.at[slot, pl.ds(0, hw)],
                sems.at[slot]).wait()

    @pl.when(r == 0)
    def _():
        issue(0)

    @pl.when(r + 1 < n_tiles)
    def _():
        issue(r + 1)

    wait(r)

    slot = jax.lax.rem(r, 2)
    xc = buf[slot].astype(jnp.bfloat16)
    zcol = jnp.zeros((hw, 1, kdim), jnp.bfloat16)
    srcs = (
        jnp.concatenate([zcol, xc[:, :w - 1]], axis=1),
        xc,
        jnp.concatenate([xc[:, 1:], zcol], axis=1),
    )
    acc = jnp.zeros((th * w, n), jnp.float32)
    for j in range(3):
        for i in range(3):
            a = srcs[j][i:i + th].reshape(th * w, kdim)
            acc += jnp.dot(a, wt_ref[i, j], preferred_element_type=jnp.float32)
    out_ref[...] = (acc + rb_ref[0, :][None, :]).reshape(th, w, n)


def kernel(inputs, kernel, bias):
    B, H, W, G_, C = inputs.shape
    kh, kw, gcin, cout = kernel.shape
    n_out = G * cout

    x = inputs.reshape(H, W, G_ * C)

    wt, rb = pl.pallas_call(
        _transform_body,
        out_shape=(
            jax.ShapeDtypeStruct((kh, kw, gcin, n_out), jnp.bfloat16),
            jax.ShapeDtypeStruct((1, n_out), jnp.float32),
        ),
    )(kernel, bias.reshape(1, cout))

    TH = 16
    n_tiles = H // TH

    conv = pl.pallas_call(
        lambda x_hbm, wt_ref, rb_ref, out_ref, buf, sems: _conv_body(
            x_hbm, wt_ref, rb_ref, out_ref, buf, sems,
            th=TH, w=W, n_tiles=n_tiles),
        grid=(n_tiles,),
        in_specs=[
            pl.BlockSpec(memory_space=pltpu.MemorySpace.HBM),
            pl.BlockSpec((kh, kw, gcin, n_out), lambda r: (0, 0, 0, 0)),
            pl.BlockSpec((1, n_out), lambda r: (0, 0)),
        ],
        out_specs=pl.BlockSpec((TH, W, n_out), lambda r: (r, 0, 0)),
        out_shape=jax.ShapeDtypeStruct((H, W, n_out), jnp.float32),
        scratch_shapes=[
            pltpu.VMEM((2, TH + 2, W, gcin), jnp.float32),
            pltpu.SemaphoreType.DMA((2,)),
        ],
        compiler_params=pltpu.CompilerParams(
            dimension_semantics=("parallel",),
        ),
    )(x, wt, rb)

    return conv.reshape(B, H, W, G, cout)


# speedup vs baseline: 1.0590x; 1.0588x over previous
import jax
import jax.numpy as jnp
from jax.experimental import pallas as pl
from jax.experimental.pallas import tpu as pltpu

G = 4


def _src_tap(s, i, j):
    if s == 0:
        return (i, j)
    if s == 1:
        return (j, 2 - i)
    if s == 2:
        return (2 - i, 2 - j)
    return (2 - j, i)


def _transform_body(k_ref, b_ref, wt_ref, rb_ref):
    gcin = k_ref.shape[2]
    cin = gcin // G
    for i in range(3):
        for j in range(3):
            cols = []
            for s in range(G):
                si, sj = _src_tap(s, i, j)
                blk = k_ref[si, sj]
                sh = s * cin
                if sh:
                    blk = jnp.concatenate([blk[gcin - sh:], blk[:gcin - sh]],
                                          axis=0)
                cols.append(blk)
            wt_ref[i, j * gcin:(j + 1) * gcin, :] = (
                jnp.concatenate(cols, axis=1).astype(jnp.bfloat16))
    rb_ref[...] = jnp.concatenate([b_ref[...]] * G, axis=-1)


def _conv_body(x_hbm, wt_ref, rb_ref, out_ref, buf, sems, *, th, w, n_tiles):
    r = pl.program_id(0)
    kdim = x_hbm.shape[2]
    n = wt_ref.shape[2]
    hw = th + 2

    def issue(t):
        slot = jax.lax.rem(t, 2)

        @pl.when(t == 0)
        def _():
            buf[slot, 0] = jnp.zeros((w, kdim), buf.dtype)
            pltpu.make_async_copy(
                x_hbm.at[pl.ds(0, th + 1)],
                buf.at[slot, pl.ds(1, th + 1)],
                sems.at[slot]).start()

        @pl.when(jnp.logical_and(t > 0, t < n_tiles - 1))
        def _():
            pltpu.make_async_copy(
                x_hbm.at[pl.ds(jnp.clip(t * th - 1, 0, x_hbm.shape[0] - hw),
                               hw)],
                buf.at[slot, pl.ds(0, hw)],
                sems.at[slot]).start()

        @pl.when(t == n_tiles - 1)
        def _():
            buf[slot, hw - 1] = jnp.zeros((w, kdim), buf.dtype)
            pltpu.make_async_copy(
                x_hbm.at[pl.ds((n_tiles - 1) * th - 1, th + 1)],
                buf.at[slot, pl.ds(0, th + 1)],
                sems.at[slot]).start()

    def wait(t):
        slot = jax.lax.rem(t, 2)
        edge = jnp.logical_or(t == 0, t == n_tiles - 1)

        @pl.when(edge)
        def _():
            pltpu.make_async_copy(
                x_hbm.at[pl.ds(0, th + 1)],
                buf.at[slot, pl.ds(1, th + 1)],
                sems.at[slot]).wait()

        @pl.when(jnp.logical_not(edge))
        def _():
            pltpu.make_async_copy(
                x_hbm.at[pl.ds(0, hw)],
                buf.at[slot, pl.ds(0, hw)],
                sems.at[slot]).wait()

    @pl.when(r == 0)
    def _():
        issue(0)

    @pl.when(r + 1 < n_tiles)
    def _():
        issue(r + 1)

    wait(r)

    slot = jax.lax.rem(r, 2)
    xc = buf[slot].astype(jnp.bfloat16)
    zcol = jnp.zeros((hw, 1, kdim), jnp.bfloat16)
    bufw = jnp.concatenate([
        jnp.concatenate([zcol, xc[:, :w - 1]], axis=1),
        xc,
        jnp.concatenate([xc[:, 1:], zcol], axis=1),
    ], axis=2)
    acc = jnp.dot(bufw[0:th].reshape(th * w, 3 * kdim), wt_ref[0],
                  preferred_element_type=jnp.float32)
    for i in range(1, 3):
        acc += jnp.dot(bufw[i:i + th].reshape(th * w, 3 * kdim), wt_ref[i],
                       preferred_element_type=jnp.float32)
    out_ref[...] = (acc + rb_ref[0, :][None, :]).reshape(th, w, n)


def kernel(inputs, kernel, bias):
    B, H, W, G_, C = inputs.shape
    kh, kw, gcin, cout = kernel.shape
    n_out = G * cout

    x = inputs.reshape(H, W, G_ * C)

    wt, rb = pl.pallas_call(
        _transform_body,
        out_shape=(
            jax.ShapeDtypeStruct((kh, kw * gcin, n_out), jnp.bfloat16),
            jax.ShapeDtypeStruct((1, n_out), jnp.float32),
        ),
    )(kernel, bias.reshape(1, cout))

    TH = 16
    n_tiles = H // TH

    conv = pl.pallas_call(
        lambda x_hbm, wt_ref, rb_ref, out_ref, buf, sems: _conv_body(
            x_hbm, wt_ref, rb_ref, out_ref, buf, sems,
            th=TH, w=W, n_tiles=n_tiles),
        grid=(n_tiles,),
        in_specs=[
            pl.BlockSpec(memory_space=pltpu.MemorySpace.HBM),
            pl.BlockSpec((kh, kw * gcin, n_out), lambda r: (0, 0, 0)),
            pl.BlockSpec((1, n_out), lambda r: (0, 0)),
        ],
        out_specs=pl.BlockSpec((TH, W, n_out), lambda r: (r, 0, 0)),
        out_shape=jax.ShapeDtypeStruct((H, W, n_out), jnp.float32),
        scratch_shapes=[
            pltpu.VMEM((2, TH + 2, W, gcin), jnp.float32),
            pltpu.SemaphoreType.DMA((2,)),
        ],
        compiler_params=pltpu.CompilerParams(
            dimension_semantics=("parallel",),
        ),
    )(x, wt, rb)

    return conv.reshape(B, H, W, G, cout)
